# baseline (speedup 1.0000x reference)
import jax
import jax.numpy as jnp
from jax.experimental import pallas as pl

_B, _N = 4, 512
_IN_NF, _IN_EF, _OUT_F = 16, 8, 16
_TI = 128
_G = 8


def _fused_kernel(wt_ref, a_ref, xt_ref, bd1_ref, bd2_ref, nw_ref,
                  wout_ref, xout_ref):
    f32 = jnp.float32
    bf16 = jnp.bfloat16
    bd1 = bd1_ref[...].astype(bf16)
    bd2 = bd2_ref[...].astype(bf16)
    wn1t = nw_ref[:, 0:16]
    wn2t = nw_ref[:, 16:32]
    be1c = nw_ref[:, 32:33]
    be2c = nw_ref[:, 33:34]
    bn1c = nw_ref[:, 34:35]
    bn2c = nw_ref[:, 35:36]
    be1 = jnp.broadcast_to(be1c[None], (_G, _OUT_F, 1)).reshape(128, 1)
    be2 = jnp.broadcast_to(be2c[None], (_G, _OUT_F, 1)).reshape(128, 1)

    xt = xt_ref[0]
    h1 = jnp.maximum(
        jnp.dot(wn1t, xt, preferred_element_type=f32) + bn1c, 0.0)
    x1t = jnp.maximum(
        jnp.dot(wn2t, h1, preferred_element_type=f32) + bn2c, 0.0)

    wtb = wt_ref[0].astype(bf16)
    hs = []
    for g in range(_TI // _G):
        rhs = wtb[g * _G:(g + 1) * _G].reshape(_G * _IN_EF, _N)
        h = jnp.maximum(
            jnp.dot(bd1, rhs, preferred_element_type=f32)
            + be1, 0.0)
        hs.append(h.astype(bf16))
    for g in range(_TI // _G):
        w2 = jnp.maximum(
            jnp.dot(bd2, hs[g], preferred_element_type=f32)
            + be2, 0.0)
        wout_ref[0, g * _G:(g + 1) * _G] = w2.reshape(_G, _OUT_F, _N)

    a = a_ref[0]
    asum = jnp.sum(a, axis=1, keepdims=True)
    inv = jnp.where(asum == 0.0, 0.0, 1.0 / asum)
    an = a * inv
    wall = wout_ref[0]
    p = wall * x1t[None] * an[:, None, :]
    xnew = jnp.sum(p, axis=2)
    xout_ref[0] = xnew


@jax.jit
def kernel(A, W, x, We1, be1, We2, be2, Wn1, bn1, Wn2, bn2):
    f32 = jnp.float32
    wt = jnp.transpose(W, (0, 1, 3, 2))
    xt = jnp.transpose(x, (0, 2, 1))

    tin = jnp.zeros((64, 16), f32)
    tin = (tin.at[0:_IN_EF].set(We1[:_IN_EF])
              .at[8:24].set(We2)
              .at[24:40].set(Wn1)
              .at[40:56].set(Wn2)
              .at[56].set(be1).at[57].set(be2)
              .at[58].set(bn1).at[59].set(bn2))
    t = tin.T
    we1t = t[:, 0:_IN_EF]
    we2t = t[:, 8:24]
    nw = jnp.zeros((16, 40), f32)
    nw = (nw.at[:, 0:16].set(t[:, 24:40])
            .at[:, 16:32].set(t[:, 40:56])
            .at[:, 32:36].set(t[:, 56:60]))
    bd1 = jnp.zeros((_G * _OUT_F, _G * _IN_EF), f32)
    bd2 = jnp.zeros((_G * _OUT_F, _G * _OUT_F), f32)
    for i in range(_G):
        bd1 = bd1.at[i * _OUT_F:(i + 1) * _OUT_F,
                     i * _IN_EF:(i + 1) * _IN_EF].set(we1t)
        bd2 = bd2.at[i * _OUT_F:(i + 1) * _OUT_F,
                     i * _OUT_F:(i + 1) * _OUT_F].set(we2t)

    const = lambda *shape: pl.BlockSpec(shape, lambda b, i: (0,) * len(shape))
    wout, xout = pl.pallas_call(
        _fused_kernel,
        grid=(_B, _N // _TI),
        in_specs=[
            pl.BlockSpec((1, _TI, _IN_EF, _N), lambda b, i: (b, i, 0, 0)),
            pl.BlockSpec((1, _TI, _N), lambda b, i: (b, i, 0)),
            pl.BlockSpec((1, _IN_NF, _N), lambda b, i: (b, 0, 0)),
            const(_G * _OUT_F, _G * _IN_EF),
            const(_G * _OUT_F, _G * _OUT_F),
            const(16, 40),
        ],
        out_specs=[
            pl.BlockSpec((1, _TI, _OUT_F, _N), lambda b, i: (b, i, 0, 0)),
            pl.BlockSpec((1, _TI, _OUT_F), lambda b, i: (b, i, 0)),
        ],
        out_shape=[
            jax.ShapeDtypeStruct((_B, _N, _OUT_F, _N), f32),
            jax.ShapeDtypeStruct((_B, _N, _OUT_F), f32),
        ],
    )(wt, A, xt, bd1, bd2, nw)
    return jnp.transpose(wout, (0, 1, 3, 2)), xout


# speedup vs baseline: 1.3259x; 1.3259x over previous
import jax
import jax.numpy as jnp
from jax.experimental import pallas as pl

_B, _N = 4, 512
_IN_NF, _IN_EF, _OUT_F = 16, 8, 16
_TI = 128
_G = 8


def _fused_kernel(wt_ref, a_ref, xt_ref, p_ref, wout_ref, xout_ref):
    bf16 = jnp.bfloat16
    bd1 = p_ref[0:128, 0:_G * _IN_EF].astype(bf16)
    bd2 = p_ref[128:256, :].astype(bf16)
    be1 = p_ref[256:384, 0:1]
    be2 = p_ref[256:384, 1:2]
    wn1t = p_ref[384:400, 0:_IN_NF]
    wn2t = p_ref[400:416, 0:_OUT_F]
    bn1 = p_ref[416:432, 0:1]
    bn2 = p_ref[416:432, 1:2]

    xt = xt_ref[0]
    h1 = jnp.maximum(
        jnp.dot(wn1t, xt, preferred_element_type=jnp.float32) + bn1, 0.0)
    x1t = jnp.maximum(
        jnp.dot(wn2t, h1, preferred_element_type=jnp.float32) + bn2, 0.0)

    wtb = wt_ref[0].astype(bf16)
    hs = []
    for g in range(_TI // _G):
        rhs = wtb[g * _G:(g + 1) * _G].reshape(_G * _IN_EF, _N)
        h = jnp.maximum(
            jnp.dot(bd1, rhs, preferred_element_type=jnp.float32)
            + be1, 0.0)
        hs.append(h.astype(bf16))
    for g in range(_TI // _G):
        w2 = jnp.maximum(
            jnp.dot(bd2, hs[g], preferred_element_type=jnp.float32)
            + be2, 0.0)
        wout_ref[0, g * _G:(g + 1) * _G] = w2.reshape(_G, _OUT_F, _N)

    a = a_ref[0]
    asum = jnp.sum(a, axis=1, keepdims=True)
    inv = jnp.where(asum == 0.0, 0.0, 1.0 / asum)
    an = a * inv
    wall = wout_ref[0]
    p = wall * x1t[None] * an[:, None, :]
    xnew = jnp.sum(p, axis=2)
    xout_ref[0] = jnp.transpose(xnew)


@jax.jit
def kernel(A, W, x, We1, be1, We2, be2, Wn1, bn1, Wn2, bn2):
    f32 = jnp.float32
    wt = jnp.transpose(W, (0, 1, 3, 2))
    xt = jnp.transpose(x, (0, 2, 1))

    eye = jnp.eye(_G, dtype=f32)
    bd1 = jnp.kron(eye, We1[:_IN_EF].T)
    bd2 = jnp.kron(eye, We2.T)
    pad = jnp.zeros((128, 128 - _G * _IN_EF), f32)
    rows_bd1 = jnp.concatenate([bd1, pad], axis=1)
    bias_cols = jnp.stack([jnp.tile(be1, _G), jnp.tile(be2, _G)], axis=1)
    rows_bias = jnp.concatenate(
        [bias_cols, jnp.zeros((128, 126), f32)], axis=1)
    rows_wn1 = jnp.concatenate(
        [Wn1.T, jnp.zeros((_IN_NF, 112), f32)], axis=1)
    rows_wn2 = jnp.concatenate(
        [Wn2.T, jnp.zeros((_OUT_F, 112), f32)], axis=1)
    nb_cols = jnp.stack([bn1, bn2], axis=1)
    rows_nb = jnp.concatenate(
        [nb_cols, jnp.zeros((16, 126), f32)], axis=1)
    params = jnp.concatenate(
        [rows_bd1, bd2, rows_bias, rows_wn1, rows_wn2, rows_nb,
         jnp.zeros((8, 128), f32)], axis=0)

    const = lambda *shape: pl.BlockSpec(shape, lambda b, i: (0,) * len(shape))
    wout, xout = pl.pallas_call(
        _fused_kernel,
        grid=(_B, _N // _TI),
        in_specs=[
            pl.BlockSpec((1, _TI, _IN_EF, _N), lambda b, i: (b, i, 0, 0)),
            pl.BlockSpec((1, _TI, _N), lambda b, i: (b, i, 0)),
            pl.BlockSpec((1, _IN_NF, _N), lambda b, i: (b, 0, 0)),
            const(440, 128),
        ],
        out_specs=[
            pl.BlockSpec((1, _TI, _OUT_F, _N), lambda b, i: (b, i, 0, 0)),
            pl.BlockSpec((1, _OUT_F, _TI), lambda b, i: (b, 0, i)),
        ],
        out_shape=[
            jax.ShapeDtypeStruct((_B, _N, _OUT_F, _N), f32),
            jax.ShapeDtypeStruct((_B, _OUT_F, _N), f32),
        ],
    )(wt, A, xt, params)
    return jnp.transpose(wout, (0, 1, 3, 2)), jnp.transpose(xout, (0, 2, 1))


# speedup vs baseline: 1.4294x; 1.0781x over previous
import jax
import jax.numpy as jnp
from jax.experimental import pallas as pl

_B, _N = 4, 512
_IN_NF, _IN_EF, _OUT_F = 16, 8, 16
_TI = 256
_G = 8


def _fused_kernel(wt_ref, a_ref, xt_ref, p_ref, wout_ref, xout_ref):
    bf16 = jnp.bfloat16
    bd1 = p_ref[0:128, 0:_G * _IN_EF].astype(bf16)
    bd2 = p_ref[128:256, :].astype(bf16)
    be1 = p_ref[256:384, 0:1]
    be2 = p_ref[256:384, 1:2]
    wn1t = p_ref[384:400, 0:_IN_NF]
    wn2t = p_ref[400:416, 0:_OUT_F]
    bn1 = p_ref[416:432, 0:1]
    bn2 = p_ref[416:432, 1:2]

    xt = xt_ref[0]
    h1 = jnp.maximum(
        jnp.dot(wn1t, xt, preferred_element_type=jnp.float32) + bn1, 0.0)
    x1t = jnp.maximum(
        jnp.dot(wn2t, h1, preferred_element_type=jnp.float32) + bn2, 0.0)

    wtb = wt_ref[0].astype(bf16)
    hs = []
    for g in range(_TI // _G):
        rhs = wtb[g * _G:(g + 1) * _G].reshape(_G * _IN_EF, _N)
        h = jnp.maximum(
            jnp.dot(bd1, rhs, preferred_element_type=jnp.float32)
            + be1, 0.0)
        hs.append(h.astype(bf16))
    for g in range(_TI // _G):
        w2 = jnp.maximum(
            jnp.dot(bd2, hs[g], preferred_element_type=jnp.float32)
            + be2, 0.0)
        wout_ref[0, g * _G:(g + 1) * _G] = w2.reshape(_G, _OUT_F, _N)

    a = a_ref[0]
    asum = jnp.sum(a, axis=1, keepdims=True)
    inv = jnp.where(asum == 0.0, 0.0, 1.0 / asum)
    an = a * inv
    wall = wout_ref[0]
    p = wall * x1t[None] * an[:, None, :]
    xnew = jnp.sum(p, axis=2)
    xout_ref[0] = jnp.transpose(xnew)


@jax.jit
def kernel(A, W, x, We1, be1, We2, be2, Wn1, bn1, Wn2, bn2):
    f32 = jnp.float32
    wt = jnp.transpose(W, (0, 1, 3, 2))
    xt = jnp.transpose(x, (0, 2, 1))

    eye = jnp.eye(_G, dtype=f32)
    bd1 = jnp.kron(eye, We1[:_IN_EF].T)
    bd2 = jnp.kron(eye, We2.T)
    pad = jnp.zeros((128, 128 - _G * _IN_EF), f32)
    rows_bd1 = jnp.concatenate([bd1, pad], axis=1)
    bias_cols = jnp.stack([jnp.tile(be1, _G), jnp.tile(be2, _G)], axis=1)
    rows_bias = jnp.concatenate(
        [bias_cols, jnp.zeros((128, 126), f32)], axis=1)
    rows_wn1 = jnp.concatenate(
        [Wn1.T, jnp.zeros((_IN_NF, 112), f32)], axis=1)
    rows_wn2 = jnp.concatenate(
        [Wn2.T, jnp.zeros((_OUT_F, 112), f32)], axis=1)
    nb_cols = jnp.stack([bn1, bn2], axis=1)
    rows_nb = jnp.concatenate(
        [nb_cols, jnp.zeros((16, 126), f32)], axis=1)
    params = jnp.concatenate(
        [rows_bd1, bd2, rows_bias, rows_wn1, rows_wn2, rows_nb,
         jnp.zeros((8, 128), f32)], axis=0)

    const = lambda *shape: pl.BlockSpec(shape, lambda b, i: (0,) * len(shape))
    wout, xout = pl.pallas_call(
        _fused_kernel,
        grid=(_B, _N // _TI),
        in_specs=[
            pl.BlockSpec((1, _TI, _IN_EF, _N), lambda b, i: (b, i, 0, 0)),
            pl.BlockSpec((1, _TI, _N), lambda b, i: (b, i, 0)),
            pl.BlockSpec((1, _IN_NF, _N), lambda b, i: (b, 0, 0)),
            const(440, 128),
        ],
        out_specs=[
            pl.BlockSpec((1, _TI, _OUT_F, _N), lambda b, i: (b, i, 0, 0)),
            pl.BlockSpec((1, _OUT_F, _TI), lambda b, i: (b, 0, i)),
        ],
        out_shape=[
            jax.ShapeDtypeStruct((_B, _N, _OUT_F, _N), f32),
            jax.ShapeDtypeStruct((_B, _OUT_F, _N), f32),
        ],
    )(wt, A, xt, params)
    return jnp.transpose(wout, (0, 1, 3, 2)), jnp.transpose(xout, (0, 2, 1))


# speedup vs baseline: 1.4309x; 1.0011x over previous
import jax
import jax.numpy as jnp
from jax.experimental import pallas as pl
from jax.experimental.pallas import tpu as pltpu

_B, _N = 4, 512
_IN_NF, _IN_EF, _OUT_F = 16, 8, 16
_TI = 256
_G = 8


def _fused_kernel(wt_ref, a_ref, xt_ref, p_ref, wout_ref, xout_ref):
    bf16 = jnp.bfloat16
    bd1 = p_ref[0:128, 0:_G * _IN_EF].astype(bf16)
    bd2 = p_ref[128:256, :].astype(bf16)
    be1 = p_ref[256:384, 0:1]
    be2 = p_ref[256:384, 1:2]
    wn1t = p_ref[384:400, 0:_IN_NF]
    wn2t = p_ref[400:416, 0:_OUT_F]
    bn1 = p_ref[416:432, 0:1]
    bn2 = p_ref[416:432, 1:2]

    xt = xt_ref[0]
    h1 = jnp.maximum(
        jnp.dot(wn1t, xt, preferred_element_type=jnp.float32) + bn1, 0.0)
    x1t = jnp.maximum(
        jnp.dot(wn2t, h1, preferred_element_type=jnp.float32) + bn2, 0.0)

    wtb = wt_ref[0].astype(bf16)
    hs = []
    for g in range(_TI // _G):
        rhs = wtb[g * _G:(g + 1) * _G].reshape(_G * _IN_EF, _N)
        h = jnp.maximum(
            jnp.dot(bd1, rhs, preferred_element_type=jnp.float32)
            + be1, 0.0)
        hs.append(h.astype(bf16))
    for g in range(_TI // _G):
        w2 = jnp.maximum(
            jnp.dot(bd2, hs[g], preferred_element_type=jnp.float32)
            + be2, 0.0)
        wout_ref[0, g * _G:(g + 1) * _G] = w2.reshape(_G, _OUT_F, _N)

    a = a_ref[0]
    asum = jnp.sum(a, axis=1, keepdims=True)
    inv = jnp.where(asum == 0.0, 0.0, 1.0 / asum)
    an = a * inv
    wall = wout_ref[0]
    p = wall * x1t[None] * an[:, None, :]
    xnew = jnp.sum(p, axis=2)
    xout_ref[0] = jnp.transpose(xnew)


@jax.jit
def kernel(A, W, x, We1, be1, We2, be2, Wn1, bn1, Wn2, bn2):
    f32 = jnp.float32
    wt = jnp.transpose(W, (0, 1, 3, 2))
    xt = jnp.transpose(x, (0, 2, 1))

    eye = jnp.eye(_G, dtype=f32)
    bd1 = jnp.kron(eye, We1[:_IN_EF].T)
    bd2 = jnp.kron(eye, We2.T)
    pad = jnp.zeros((128, 128 - _G * _IN_EF), f32)
    rows_bd1 = jnp.concatenate([bd1, pad], axis=1)
    bias_cols = jnp.stack([jnp.tile(be1, _G), jnp.tile(be2, _G)], axis=1)
    rows_bias = jnp.concatenate(
        [bias_cols, jnp.zeros((128, 126), f32)], axis=1)
    rows_wn1 = jnp.concatenate(
        [Wn1.T, jnp.zeros((_IN_NF, 112), f32)], axis=1)
    rows_wn2 = jnp.concatenate(
        [Wn2.T, jnp.zeros((_OUT_F, 112), f32)], axis=1)
    nb_cols = jnp.stack([bn1, bn2], axis=1)
    rows_nb = jnp.concatenate(
        [nb_cols, jnp.zeros((16, 126), f32)], axis=1)
    params = jnp.concatenate(
        [rows_bd1, bd2, rows_bias, rows_wn1, rows_wn2, rows_nb,
         jnp.zeros((8, 128), f32)], axis=0)

    const = lambda *shape: pl.BlockSpec(shape, lambda b, i: (0,) * len(shape))
    wout, xout = pl.pallas_call(
        _fused_kernel,
        grid=(_B, _N // _TI),
        in_specs=[
            pl.BlockSpec((1, _TI, _IN_EF, _N), lambda b, i: (b, i, 0, 0)),
            pl.BlockSpec((1, _TI, _N), lambda b, i: (b, i, 0)),
            pl.BlockSpec((1, _IN_NF, _N), lambda b, i: (b, 0, 0)),
            const(440, 128),
        ],
        out_specs=[
            pl.BlockSpec((1, _TI, _OUT_F, _N), lambda b, i: (b, i, 0, 0)),
            pl.BlockSpec((1, _OUT_F, _TI), lambda b, i: (b, 0, i)),
        ],
        out_shape=[
            jax.ShapeDtypeStruct((_B, _N, _OUT_F, _N), f32),
            jax.ShapeDtypeStruct((_B, _OUT_F, _N), f32),
        ],
        compiler_params=pltpu.CompilerParams(
            dimension_semantics=("parallel", "parallel")),
    )(wt, A, xt, params)
    return jnp.transpose(wout, (0, 1, 3, 2)), jnp.transpose(xout, (0, 2, 1))
